# baseline (device time: 48652 ns/iter reference)
import jax
import jax.numpy as jnp
from jax import lax
from jax.experimental import pallas as pl
from jax.experimental.pallas import tpu as pltpu

QROWS = 1024
C = 128
NC = QROWS // C


def kernel(x):
    m_per, n = x.shape

    def body(x_ref, out_ref, xstage, mine_ref, recv_ref,
             y_send, y_recv, x_send, x_recv, z_send, z_recv,
             load_sem, store_sem):
        my_x = lax.axis_index("x")
        my_y = lax.axis_index("y")
        my_z = lax.axis_index("z")
        y_nbr = (my_x, 1 - my_y, my_z)
        x_nbr = (1 - my_x, my_y, my_z)
        z_nbr = (my_x, my_y, 1 - my_z)

        qp = 2 * my_x + my_z
        qx = 2 * (1 - my_x) + my_z
        qz = 2 * my_x + (1 - my_z)
        qd = 2 * (1 - my_x) + (1 - my_z)

        quarters = [qp, qd, qx, qz]
        loads = []
        for j, q in enumerate(quarters):
            ld = pltpu.make_async_copy(
                x_ref.at[pl.ds(q * QROWS, QROWS), :],
                xstage.at[pl.ds(q * QROWS, QROWS), :],
                load_sem.at[j],
            )
            ld.start()
            loads.append(ld)

        barrier_sem = pltpu.get_barrier_semaphore()
        for nbr in (y_nbr, x_nbr, z_nbr):
            pl.semaphore_signal(
                barrier_sem, inc=1, device_id=nbr,
                device_id_type=pl.DeviceIdType.MESH,
            )
        pl.semaphore_wait(barrier_sem, 3)

        own = my_y * m_per
        other = (1 - my_y) * m_per

        def rdma(src_ref_, row, nrows, ssem, rsem, nbr):
            return pltpu.make_async_remote_copy(
                src_ref=src_ref_.at[pl.ds(row, nrows), :],
                dst_ref=recv_ref.at[pl.ds(row, nrows), :],
                send_sem=ssem,
                recv_sem=rsem,
                device_id=nbr,
                device_id_type=pl.DeviceIdType.MESH,
            )

        sends = []

        loads[0].wait()
        for c in range(NC):
            row = qp * QROWS + c * C
            mine_ref[pl.ds(row, C), :] = (
                xstage[pl.ds(row, C), :].astype(jnp.bfloat16)
            )
            r = rdma(mine_ref, row, C, y_send.at[c], y_recv.at[c], y_nbr)
            r.start()
            sends.append(r)
        loads[1].wait()
        for i, c in enumerate((NC - 2, NC - 1)):
            row = qd * QROWS + c * C
            mine_ref[pl.ds(row, C), :] = (
                xstage[pl.ds(row, C), :].astype(jnp.bfloat16)
            )
            r = rdma(mine_ref, row, C,
                     y_send.at[NC + i], y_recv.at[NC + i], y_nbr)
            r.start()
            sends.append(r)

        row_d = qd * QROWS
        mine_ref[pl.ds(row_d, QROWS - 2 * C), :] = (
            xstage[pl.ds(row_d, QROWS - 2 * C), :].astype(jnp.bfloat16)
        )
        for j in (2, 3):
            loads[j].wait()
            row = quarters[j] * QROWS
            mine_ref[pl.ds(row, QROWS), :] = (
                xstage[pl.ds(row, QROWS), :].astype(jnp.bfloat16)
            )
        mine_store = pltpu.make_async_copy(
            mine_ref, out_ref.at[pl.ds(own, m_per), :], store_sem.at[0]
        )
        mine_store.start()

        for c in range(NC):
            row = qp * QROWS + c * C
            rdma(recv_ref, row, C, y_send.at[c], y_recv.at[c],
                 y_nbr).wait_recv()
            rx = rdma(recv_ref, row, C, x_send.at[c], x_recv.at[c], x_nbr)
            rx.start()
            sends.append(rx)
            rz = rdma(recv_ref, row, C, z_send.at[c], z_recv.at[c], z_nbr)
            rz.start()
            sends.append(rz)
        st_p = pltpu.make_async_copy(
            recv_ref.at[pl.ds(qp * QROWS, QROWS), :],
            out_ref.at[pl.ds(other + qp * QROWS, QROWS), :],
            store_sem.at[1],
        )
        st_p.start()

        for c in range(3):
            row = qz * QROWS + c * C
            rdma(recv_ref, row, C, z_send.at[c], z_recv.at[c],
                 z_nbr).wait_recv()
            r = rdma(recv_ref, row, C,
                     x_send.at[NC + c], x_recv.at[NC + c], x_nbr)
            r.start()
            sends.append(r)

        for i, c in enumerate(range(3, 6)):
            row = qx * QROWS + c * C
            rdma(recv_ref, row, C, x_send.at[c], x_recv.at[c],
                 x_nbr).wait_recv()
            r = rdma(recv_ref, row, C,
                     z_send.at[NC + i], z_recv.at[NC + i], z_nbr)
            r.start()
            sends.append(r)

        for c in (0, 1, 2, 6, 7):
            row = qx * QROWS + c * C
            rdma(recv_ref, row, C, x_send.at[c], x_recv.at[c],
                 x_nbr).wait_recv()
        st_x = pltpu.make_async_copy(
            recv_ref.at[pl.ds(qx * QROWS, QROWS), :],
            out_ref.at[pl.ds(other + qx * QROWS, QROWS), :],
            store_sem.at[2],
        )
        st_x.start()
        for c in range(3, NC):
            row = qz * QROWS + c * C
            rdma(recv_ref, row, C, z_send.at[c], z_recv.at[c],
                 z_nbr).wait_recv()
        st_z = pltpu.make_async_copy(
            recv_ref.at[pl.ds(qz * QROWS, QROWS), :],
            out_ref.at[pl.ds(other + qz * QROWS, QROWS), :],
            store_sem.at[3],
        )
        st_z.start()
        for c in range(3):
            row = qd * QROWS + c * C
            rdma(recv_ref, row, C, x_send.at[NC + c], x_recv.at[NC + c],
                 x_nbr).wait_recv()
        for i, c in enumerate(range(3, 6)):
            row = qd * QROWS + c * C
            rdma(recv_ref, row, C, z_send.at[NC + i], z_recv.at[NC + i],
                 z_nbr).wait_recv()
        for i, c in enumerate((NC - 2, NC - 1)):
            row = qd * QROWS + c * C
            rdma(recv_ref, row, C, y_send.at[NC + i], y_recv.at[NC + i],
                 y_nbr).wait_recv()
        st_d = pltpu.make_async_copy(
            recv_ref.at[pl.ds(qd * QROWS, QROWS), :],
            out_ref.at[pl.ds(other + qd * QROWS, QROWS), :],
            store_sem.at[4],
        )
        st_d.start()

        for r in sends:
            r.wait_send()
        mine_store.wait()
        st_p.wait()
        st_x.wait()
        st_z.wait()
        st_d.wait()

    return pl.pallas_call(
        body,
        out_shape=jax.ShapeDtypeStruct((2 * m_per, n), jnp.bfloat16),
        in_specs=[pl.BlockSpec(memory_space=pltpu.MemorySpace.HBM)],
        out_specs=pl.BlockSpec(memory_space=pltpu.MemorySpace.HBM),
        scratch_shapes=[
            pltpu.VMEM((m_per, n), jnp.float32),
            pltpu.VMEM((m_per, n), jnp.bfloat16),
            pltpu.VMEM((m_per, n), jnp.bfloat16),
            pltpu.SemaphoreType.DMA((NC + 2,)),
            pltpu.SemaphoreType.DMA((NC + 2,)),
            pltpu.SemaphoreType.DMA((NC + 3,)),
            pltpu.SemaphoreType.DMA((NC + 3,)),
            pltpu.SemaphoreType.DMA((NC + 3,)),
            pltpu.SemaphoreType.DMA((NC + 3,)),
            pltpu.SemaphoreType.DMA((4,)),
            pltpu.SemaphoreType.DMA((5,)),
        ],
        compiler_params=pltpu.CompilerParams(collective_id=0),
    )(x)


# device time: 48008 ns/iter; 1.0134x vs baseline; 1.0134x over previous
import jax
import jax.numpy as jnp
from jax import lax
from jax.experimental import pallas as pl
from jax.experimental.pallas import tpu as pltpu

QROWS = 1024
C = 128
NC = QROWS // C


def kernel(x):
    m_per, n = x.shape

    def body(x_ref, out_ref, xstage, mine_ref, recv_ref,
             y_send, y_recv, x_send, x_recv, z_send, z_recv,
             load_sem, store_sem):
        my_x = lax.axis_index("x")
        my_y = lax.axis_index("y")
        my_z = lax.axis_index("z")
        y_nbr = (my_x, 1 - my_y, my_z)
        x_nbr = (1 - my_x, my_y, my_z)
        z_nbr = (my_x, my_y, 1 - my_z)

        qp = 2 * my_x + my_z
        qx = 2 * (1 - my_x) + my_z
        qz = 2 * my_x + (1 - my_z)
        qd = 2 * (1 - my_x) + (1 - my_z)

        chunk_loads = []
        for c in range(NC):
            row = qp * QROWS + c * C
            ld = pltpu.make_async_copy(
                x_ref.at[pl.ds(row, C), :],
                xstage.at[pl.ds(row, C), :],
                load_sem.at[c],
            )
            ld.start()
            chunk_loads.append(ld)
        quarters = [qp, qd, qx, qz]
        loads = [None]
        for j, q in enumerate(quarters[1:], start=1):
            ld = pltpu.make_async_copy(
                x_ref.at[pl.ds(q * QROWS, QROWS), :],
                xstage.at[pl.ds(q * QROWS, QROWS), :],
                load_sem.at[NC + j - 1],
            )
            ld.start()
            loads.append(ld)

        barrier_sem = pltpu.get_barrier_semaphore()
        for nbr in (y_nbr, x_nbr, z_nbr):
            pl.semaphore_signal(
                barrier_sem, inc=1, device_id=nbr,
                device_id_type=pl.DeviceIdType.MESH,
            )
        pl.semaphore_wait(barrier_sem, 3)

        own = my_y * m_per
        other = (1 - my_y) * m_per

        def rdma(src_ref_, row, nrows, ssem, rsem, nbr):
            return pltpu.make_async_remote_copy(
                src_ref=src_ref_.at[pl.ds(row, nrows), :],
                dst_ref=recv_ref.at[pl.ds(row, nrows), :],
                send_sem=ssem,
                recv_sem=rsem,
                device_id=nbr,
                device_id_type=pl.DeviceIdType.MESH,
            )

        sends = []

        for c in range(NC):
            row = qp * QROWS + c * C
            chunk_loads[c].wait()
            mine_ref[pl.ds(row, C), :] = (
                xstage[pl.ds(row, C), :].astype(jnp.bfloat16)
            )
            r = rdma(mine_ref, row, C, y_send.at[c], y_recv.at[c], y_nbr)
            r.start()
            sends.append(r)
        loads[1].wait()
        for i, c in enumerate((NC - 2, NC - 1)):
            row = qd * QROWS + c * C
            mine_ref[pl.ds(row, C), :] = (
                xstage[pl.ds(row, C), :].astype(jnp.bfloat16)
            )
            r = rdma(mine_ref, row, C,
                     y_send.at[NC + i], y_recv.at[NC + i], y_nbr)
            r.start()
            sends.append(r)

        row_d = qd * QROWS
        mine_ref[pl.ds(row_d, QROWS - 2 * C), :] = (
            xstage[pl.ds(row_d, QROWS - 2 * C), :].astype(jnp.bfloat16)
        )
        for j in (2, 3):
            loads[j].wait()
            row = quarters[j] * QROWS
            mine_ref[pl.ds(row, QROWS), :] = (
                xstage[pl.ds(row, QROWS), :].astype(jnp.bfloat16)
            )
        mine_store = pltpu.make_async_copy(
            mine_ref, out_ref.at[pl.ds(own, m_per), :], store_sem.at[0]
        )
        mine_store.start()

        for c in range(NC):
            row = qp * QROWS + c * C
            rdma(recv_ref, row, C, y_send.at[c], y_recv.at[c],
                 y_nbr).wait_recv()
            rx = rdma(recv_ref, row, C, x_send.at[c], x_recv.at[c], x_nbr)
            rx.start()
            sends.append(rx)
            rz = rdma(recv_ref, row, C, z_send.at[c], z_recv.at[c], z_nbr)
            rz.start()
            sends.append(rz)
        st_p = pltpu.make_async_copy(
            recv_ref.at[pl.ds(qp * QROWS, QROWS), :],
            out_ref.at[pl.ds(other + qp * QROWS, QROWS), :],
            store_sem.at[1],
        )
        st_p.start()

        for c in range(3):
            row = qz * QROWS + c * C
            rdma(recv_ref, row, C, z_send.at[c], z_recv.at[c],
                 z_nbr).wait_recv()
            r = rdma(recv_ref, row, C,
                     x_send.at[NC + c], x_recv.at[NC + c], x_nbr)
            r.start()
            sends.append(r)

        for i, c in enumerate(range(3, 6)):
            row = qx * QROWS + c * C
            rdma(recv_ref, row, C, x_send.at[c], x_recv.at[c],
                 x_nbr).wait_recv()
            r = rdma(recv_ref, row, C,
                     z_send.at[NC + i], z_recv.at[NC + i], z_nbr)
            r.start()
            sends.append(r)

        for c in (0, 1, 2, 6, 7):
            row = qx * QROWS + c * C
            rdma(recv_ref, row, C, x_send.at[c], x_recv.at[c],
                 x_nbr).wait_recv()
        st_x = pltpu.make_async_copy(
            recv_ref.at[pl.ds(qx * QROWS, QROWS), :],
            out_ref.at[pl.ds(other + qx * QROWS, QROWS), :],
            store_sem.at[2],
        )
        st_x.start()
        for c in range(3, NC):
            row = qz * QROWS + c * C
            rdma(recv_ref, row, C, z_send.at[c], z_recv.at[c],
                 z_nbr).wait_recv()
        st_z = pltpu.make_async_copy(
            recv_ref.at[pl.ds(qz * QROWS, QROWS), :],
            out_ref.at[pl.ds(other + qz * QROWS, QROWS), :],
            store_sem.at[3],
        )
        st_z.start()
        for c in range(3):
            row = qd * QROWS + c * C
            rdma(recv_ref, row, C, x_send.at[NC + c], x_recv.at[NC + c],
                 x_nbr).wait_recv()
        for i, c in enumerate(range(3, 6)):
            row = qd * QROWS + c * C
            rdma(recv_ref, row, C, z_send.at[NC + i], z_recv.at[NC + i],
                 z_nbr).wait_recv()
        for i, c in enumerate((NC - 2, NC - 1)):
            row = qd * QROWS + c * C
            rdma(recv_ref, row, C, y_send.at[NC + i], y_recv.at[NC + i],
                 y_nbr).wait_recv()
        st_d = pltpu.make_async_copy(
            recv_ref.at[pl.ds(qd * QROWS, QROWS), :],
            out_ref.at[pl.ds(other + qd * QROWS, QROWS), :],
            store_sem.at[4],
        )
        st_d.start()

        for r in sends:
            r.wait_send()
        mine_store.wait()
        st_p.wait()
        st_x.wait()
        st_z.wait()
        st_d.wait()

    return pl.pallas_call(
        body,
        out_shape=jax.ShapeDtypeStruct((2 * m_per, n), jnp.bfloat16),
        in_specs=[pl.BlockSpec(memory_space=pltpu.MemorySpace.HBM)],
        out_specs=pl.BlockSpec(memory_space=pltpu.MemorySpace.HBM),
        scratch_shapes=[
            pltpu.VMEM((m_per, n), jnp.float32),
            pltpu.VMEM((m_per, n), jnp.bfloat16),
            pltpu.VMEM((m_per, n), jnp.bfloat16),
            pltpu.SemaphoreType.DMA((NC + 2,)),
            pltpu.SemaphoreType.DMA((NC + 2,)),
            pltpu.SemaphoreType.DMA((NC + 3,)),
            pltpu.SemaphoreType.DMA((NC + 3,)),
            pltpu.SemaphoreType.DMA((NC + 3,)),
            pltpu.SemaphoreType.DMA((NC + 3,)),
            pltpu.SemaphoreType.DMA((NC + 3,)),
            pltpu.SemaphoreType.DMA((5,)),
        ],
        compiler_params=pltpu.CompilerParams(collective_id=0),
    )(x)


# device time: 47307 ns/iter; 1.0284x vs baseline; 1.0148x over previous
import jax
import jax.numpy as jnp
from jax import lax
from jax.experimental import pallas as pl
from jax.experimental.pallas import tpu as pltpu

QROWS = 1024
C = 64
NC = QROWS // C
E_Y = 6
H_X = 5
H_Z = 5
DY = tuple(range(NC - E_Y, NC))
DX = tuple(range(H_X))
DZ = tuple(range(H_X, H_X + H_Z))


def kernel(x):
    m_per, n = x.shape

    def body(x_ref, out_ref, xstage, mine_ref, recv_ref,
             y_send, y_recv, x_send, x_recv, z_send, z_recv,
             load_sem, store_sem):
        my_x = lax.axis_index("x")
        my_y = lax.axis_index("y")
        my_z = lax.axis_index("z")
        y_nbr = (my_x, 1 - my_y, my_z)
        x_nbr = (1 - my_x, my_y, my_z)
        z_nbr = (my_x, my_y, 1 - my_z)

        qp = 2 * my_x + my_z
        qx = 2 * (1 - my_x) + my_z
        qz = 2 * my_x + (1 - my_z)
        qd = 2 * (1 - my_x) + (1 - my_z)

        chunk_loads = []
        for c in range(NC):
            row = qp * QROWS + c * C
            ld = pltpu.make_async_copy(
                x_ref.at[pl.ds(row, C), :],
                xstage.at[pl.ds(row, C), :],
                load_sem.at[c],
            )
            ld.start()
            chunk_loads.append(ld)
        quarters = [qp, qd, qx, qz]
        loads = [None]
        for j, q in enumerate(quarters[1:], start=1):
            ld = pltpu.make_async_copy(
                x_ref.at[pl.ds(q * QROWS, QROWS), :],
                xstage.at[pl.ds(q * QROWS, QROWS), :],
                load_sem.at[NC + j - 1],
            )
            ld.start()
            loads.append(ld)

        barrier_sem = pltpu.get_barrier_semaphore()
        for nbr in (y_nbr, x_nbr, z_nbr):
            pl.semaphore_signal(
                barrier_sem, inc=1, device_id=nbr,
                device_id_type=pl.DeviceIdType.MESH,
            )
        pl.semaphore_wait(barrier_sem, 3)

        own = my_y * m_per
        other = (1 - my_y) * m_per

        def rdma(src_ref_, row, ssem, rsem, nbr):
            return pltpu.make_async_remote_copy(
                src_ref=src_ref_.at[pl.ds(row, C), :],
                dst_ref=recv_ref.at[pl.ds(row, C), :],
                send_sem=ssem,
                recv_sem=rsem,
                device_id=nbr,
                device_id_type=pl.DeviceIdType.MESH,
            )

        sends = []

        for c in range(NC):
            row = qp * QROWS + c * C
            chunk_loads[c].wait()
            mine_ref[pl.ds(row, C), :] = (
                xstage[pl.ds(row, C), :].astype(jnp.bfloat16)
            )
            r = rdma(mine_ref, row, y_send.at[c], y_recv.at[c], y_nbr)
            r.start()
            sends.append(r)
        loads[1].wait()
        for i, c in enumerate(DY):
            row = qd * QROWS + c * C
            mine_ref[pl.ds(row, C), :] = (
                xstage[pl.ds(row, C), :].astype(jnp.bfloat16)
            )
            r = rdma(mine_ref, row,
                     y_send.at[NC + i], y_recv.at[NC + i], y_nbr)
            r.start()
            sends.append(r)

        row_d = qd * QROWS
        mine_ref[pl.ds(row_d, QROWS - E_Y * C), :] = (
            xstage[pl.ds(row_d, QROWS - E_Y * C), :].astype(jnp.bfloat16)
        )
        for j in (2, 3):
            loads[j].wait()
            row = quarters[j] * QROWS
            mine_ref[pl.ds(row, QROWS), :] = (
                xstage[pl.ds(row, QROWS), :].astype(jnp.bfloat16)
            )
        mine_store = pltpu.make_async_copy(
            mine_ref, out_ref.at[pl.ds(own, m_per), :], store_sem.at[0]
        )
        mine_store.start()

        for c in range(NC):
            row = qp * QROWS + c * C
            rdma(recv_ref, row, y_send.at[c], y_recv.at[c],
                 y_nbr).wait_recv()
            rx = rdma(recv_ref, row, x_send.at[c], x_recv.at[c], x_nbr)
            rx.start()
            sends.append(rx)
            rz = rdma(recv_ref, row, z_send.at[c], z_recv.at[c], z_nbr)
            rz.start()
            sends.append(rz)
        st_p = pltpu.make_async_copy(
            recv_ref.at[pl.ds(qp * QROWS, QROWS), :],
            out_ref.at[pl.ds(other + qp * QROWS, QROWS), :],
            store_sem.at[1],
        )
        st_p.start()

        for i, c in enumerate(DX):
            row = qz * QROWS + c * C
            rdma(recv_ref, row, z_send.at[c], z_recv.at[c],
                 z_nbr).wait_recv()
            r = rdma(recv_ref, row,
                     x_send.at[NC + i], x_recv.at[NC + i], x_nbr)
            r.start()
            sends.append(r)

        for i, c in enumerate(DZ):
            row = qx * QROWS + c * C
            rdma(recv_ref, row, x_send.at[c], x_recv.at[c],
                 x_nbr).wait_recv()
            r = rdma(recv_ref, row,
                     z_send.at[NC + i], z_recv.at[NC + i], z_nbr)
            r.start()
            sends.append(r)

        for c in range(NC):
            if c in DZ:
                continue
            row = qx * QROWS + c * C
            rdma(recv_ref, row, x_send.at[c], x_recv.at[c],
                 x_nbr).wait_recv()
        st_x = pltpu.make_async_copy(
            recv_ref.at[pl.ds(qx * QROWS, QROWS), :],
            out_ref.at[pl.ds(other + qx * QROWS, QROWS), :],
            store_sem.at[2],
        )
        st_x.start()
        for c in range(NC):
            if c in DX:
                continue
            row = qz * QROWS + c * C
            rdma(recv_ref, row, z_send.at[c], z_recv.at[c],
                 z_nbr).wait_recv()
        st_z = pltpu.make_async_copy(
            recv_ref.at[pl.ds(qz * QROWS, QROWS), :],
            out_ref.at[pl.ds(other + qz * QROWS, QROWS), :],
            store_sem.at[3],
        )
        st_z.start()
        for i, c in enumerate(DX):
            row = qd * QROWS + c * C
            rdma(recv_ref, row, x_send.at[NC + i], x_recv.at[NC + i],
                 x_nbr).wait_recv()
        for i, c in enumerate(DZ):
            row = qd * QROWS + c * C
            rdma(recv_ref, row, z_send.at[NC + i], z_recv.at[NC + i],
                 z_nbr).wait_recv()
        for i, c in enumerate(DY):
            row = qd * QROWS + c * C
            rdma(recv_ref, row, y_send.at[NC + i], y_recv.at[NC + i],
                 y_nbr).wait_recv()
        st_d = pltpu.make_async_copy(
            recv_ref.at[pl.ds(qd * QROWS, QROWS), :],
            out_ref.at[pl.ds(other + qd * QROWS, QROWS), :],
            store_sem.at[4],
        )
        st_d.start()

        for r in sends:
            r.wait_send()
        mine_store.wait()
        st_p.wait()
        st_x.wait()
        st_z.wait()
        st_d.wait()

    return pl.pallas_call(
        body,
        out_shape=jax.ShapeDtypeStruct((2 * m_per, n), jnp.bfloat16),
        in_specs=[pl.BlockSpec(memory_space=pltpu.MemorySpace.HBM)],
        out_specs=pl.BlockSpec(memory_space=pltpu.MemorySpace.HBM),
        scratch_shapes=[
            pltpu.VMEM((m_per, n), jnp.float32),
            pltpu.VMEM((m_per, n), jnp.bfloat16),
            pltpu.VMEM((m_per, n), jnp.bfloat16),
            pltpu.SemaphoreType.DMA((NC + E_Y,)),
            pltpu.SemaphoreType.DMA((NC + E_Y,)),
            pltpu.SemaphoreType.DMA((NC + H_X,)),
            pltpu.SemaphoreType.DMA((NC + H_X,)),
            pltpu.SemaphoreType.DMA((NC + H_Z,)),
            pltpu.SemaphoreType.DMA((NC + H_Z,)),
            pltpu.SemaphoreType.DMA((NC + 3,)),
            pltpu.SemaphoreType.DMA((5,)),
        ],
        compiler_params=pltpu.CompilerParams(collective_id=0),
    )(x)
